# baseline (speedup 1.0000x reference)
import functools

import jax
import jax.numpy as jnp
from jax import lax
from jax.experimental import pallas as pl
from jax.experimental.pallas import tpu as pltpu
from jax.experimental.pallas import tpu_sc as plsc

N_CP = 240
N_J = 52
DIM = 32
BATCH = 16384

NUM_WORKERS = 32
CHUNK = 1024
N_CHUNKS = BATCH // CHUNK
UNITS = N_J * N_CHUNKS
UNITS_PER_W = UNITS // NUM_WORKERS
LANES = 16
TROW = 2 * LANES
TBL = N_CP * TROW

_GATHER_DNUMS = lax.GatherDimensionNumbers(
    offset_dims=(), collapsed_slice_dims=(0,), start_index_map=(0,))


def _splat(vec, lane):
    idx = jnp.full((LANES, 1), lane, jnp.int32)
    return lax.gather(vec, idx, _GATHER_DNUMS, slice_sizes=(1,),
                      mode=lax.GatherScatterMode.PROMISE_IN_BOUNDS)


def _pack_pairs(x):
    xb = x.astype(jnp.bfloat16)
    lo = lax.bitcast_convert_type(xb[..., :LANES], jnp.uint16).astype(jnp.uint32)
    hi = lax.bitcast_convert_type(xb[..., LANES:], jnp.uint16).astype(jnp.uint32)
    return (lo | (hi << 16)).astype(jnp.int32)


def _build_kernel():
    mesh = plsc.VectorSubcoreMesh(core_axis_name="c", subcore_axis_name="s")

    @functools.partial(
        pl.kernel,
        mesh=mesh,
        out_type=jax.ShapeDtypeStruct((N_J * BATCH * DIM,), jnp.float32),
        compiler_params=pltpu.CompilerParams(needs_layout_passes=False),
        scratch_types=[
            pltpu.VMEM((TBL,), jnp.int32),
            pltpu.VMEM((TBL,), jnp.int32),
            pltpu.VMEM((CHUNK,), jnp.float32),
            pltpu.VMEM((CHUNK,), jnp.float32),
            pltpu.VMEM((CHUNK,), jnp.int32),
            pltpu.VMEM((CHUNK,), jnp.float32),
            pltpu.VMEM((CHUNK * DIM,), jnp.float32),
            pltpu.VMEM((CHUNK * DIM,), jnp.float32),
            pltpu.SemaphoreType.DMA,
            pltpu.SemaphoreType.DMA,
            pltpu.SemaphoreType.DMA,
            pltpu.SemaphoreType.DMA,
            pltpu.SemaphoreType.DMA,
            pltpu.SemaphoreType.DMA,
        ],
    )
    def interp_kernel(t_hbm, tbl_hbm, out_hbm,
                      tbl_a, tbl_b, t_a, t_b, l_v, a_v, o_a, o_b,
                      stbl_a, stbl_b, st_a, st_b, so_a, so_b):
        wid = lax.axis_index("s") * 2 + lax.axis_index("c")
        lane_iota = lax.iota(jnp.int32, LANES)
        hi_iota = lane_iota + LANES
        tbls = (tbl_a, tbl_b)
        ts = (t_a, t_b)
        os_ = (o_a, o_b)
        stbls = (stbl_a, stbl_b)
        sts = (st_a, st_b)
        sos = (so_a, so_b)

        def unit_of(s):
            u = s * NUM_WORKERS + wid
            j = u // N_CHUNKS
            c0 = (u % N_CHUNKS) * CHUNK
            return j, c0

        def prefetch(s, b):
            j, c0 = unit_of(s)
            pltpu.async_copy(tbl_hbm.at[pl.ds(j * TBL, TBL)], tbls[b], stbls[b])
            pltpu.async_copy(t_hbm.at[pl.ds(c0, CHUNK)], ts[b], sts[b])

        def run_unit(s, b, do_drain, do_prefetch):
            j, c0 = unit_of(s)
            tbl_v, t_v, o_v = tbls[b], ts[b], os_[b]

            pltpu.make_async_copy(
                tbl_hbm.at[pl.ds(j * TBL, TBL)], tbl_v, stbls[b]).wait()
            pltpu.make_async_copy(
                t_hbm.at[pl.ds(c0, CHUNK)], t_v, sts[b]).wait()

            if isinstance(do_prefetch, bool):
                if do_prefetch:
                    prefetch(s + 1, 1 - b)
            else:
                @pl.when(do_prefetch)
                def _prefetch_next():
                    prefetch(s + 1, 1 - b)

            @plsc.parallel_loop(0, CHUNK, LANES, unroll=2)
            def idx_body(base):
                tv = t_v[pl.ds(base, LANES)]
                fi = tv * float(N_CP)
                li = fi.astype(jnp.int32)
                li = jnp.minimum(li, N_CP - 1)
                a = fi - li.astype(jnp.float32)
                l_v[pl.ds(base, LANES)] = li * TROW
                a_v[pl.ds(base, LANES)] = a

            @pl.when(do_drain)
            def _drain():
                pltpu.make_async_copy(
                    o_v,
                    out_hbm.at[pl.ds((j * BATCH + c0) * DIM, CHUNK * DIM)],
                    sos[b]).wait()

            @plsc.parallel_loop(0, CHUNK, LANES, unroll=4)
            def grp_body(base):
                lvec = l_v[pl.ds(base, LANES)]
                avec = a_v[pl.ds(base, LANES)]
                for e in range(LANES):
                    lsp = _splat(lvec, e)
                    asp = _splat(avec, e)
                    cw = plsc.load_gather(tbl_v, [lsp + lane_iota])
                    dw = plsc.load_gather(tbl_v, [lsp + hi_iota])
                    cp0, cp1 = plsc.unpack(
                        plsc.bitcast(cw, jnp.bfloat16),
                        format=plsc.PackFormat.INTERLEAVED,
                        preferred_element_type=jnp.float32)
                    d0, d1 = plsc.unpack(
                        plsc.bitcast(dw, jnp.bfloat16),
                        format=plsc.PackFormat.INTERLEAVED,
                        preferred_element_type=jnp.float32)
                    o = base * DIM + e * DIM
                    o_v[pl.ds(o, LANES)] = cp0 + asp * d0
                    o_v[pl.ds(o + LANES, LANES)] = cp1 + asp * d1

            pltpu.async_copy(
                o_v, out_hbm.at[pl.ds((j * BATCH + c0) * DIM, CHUNK * DIM)],
                sos[b])

        prefetch(0, 0)

        def pair_body(s2, carry):
            s0 = s2 * 2
            run_unit(s0, 0, s2 > 0, True)
            run_unit(s0 + 1, 1, s2 > 0, s0 + 2 < UNITS_PER_W)
            return carry

        lax.fori_loop(0, UNITS_PER_W // 2, pair_body, 0)

        for o_v, sem in ((o_a, so_a), (o_b, so_b)):
            pltpu.make_async_copy(
                o_v, out_hbm.at[pl.ds(0, CHUNK * DIM)], sem).wait()

    return interp_kernel


_INTERP = _build_kernel()


def kernel(t, control_points):
    cpt = jnp.swapaxes(control_points, 0, 1)
    delta = jnp.concatenate(
        [cpt[:, 1:, :] - cpt[:, :-1, :],
         jnp.zeros((N_J, 1, DIM), jnp.float32)], axis=1)
    tbl = jnp.concatenate([_pack_pairs(cpt), _pack_pairs(delta)], axis=-1)
    out_flat = _INTERP(t.reshape(BATCH), tbl.reshape(N_J * TBL))
    return out_flat.reshape(N_J, BATCH, DIM)


# speedup vs baseline: 1.0109x; 1.0109x over previous
import functools

import jax
import jax.numpy as jnp
from jax import lax
from jax.experimental import pallas as pl
from jax.experimental.pallas import tpu as pltpu
from jax.experimental.pallas import tpu_sc as plsc

N_CP = 240
N_J = 52
DIM = 32
BATCH = 16384

NUM_WORKERS = 32
CHUNK = 1024
N_CHUNKS = BATCH // CHUNK
UNITS = N_J * N_CHUNKS
UNITS_PER_W = UNITS // NUM_WORKERS
LANES = 16
TROW = 2 * LANES
TBL = N_CP * TROW

_GATHER_DNUMS = lax.GatherDimensionNumbers(
    offset_dims=(), collapsed_slice_dims=(0,), start_index_map=(0,))


def _splat(vec, lane):
    idx = jnp.full((LANES, 1), lane, jnp.int32)
    return lax.gather(vec, idx, _GATHER_DNUMS, slice_sizes=(1,),
                      mode=lax.GatherScatterMode.PROMISE_IN_BOUNDS)


def _pack_pairs(x):
    xb = x.astype(jnp.bfloat16)
    lo = lax.bitcast_convert_type(xb[..., :LANES], jnp.uint16).astype(jnp.uint32)
    hi = lax.bitcast_convert_type(xb[..., LANES:], jnp.uint16).astype(jnp.uint32)
    return (lo | (hi << 16)).astype(jnp.int32)


def _build_kernel():
    mesh = plsc.VectorSubcoreMesh(core_axis_name="c", subcore_axis_name="s")

    @functools.partial(
        pl.kernel,
        mesh=mesh,
        out_type=jax.ShapeDtypeStruct((N_J * BATCH * DIM,), jnp.float32),
        compiler_params=pltpu.CompilerParams(needs_layout_passes=False),
        scratch_types=[
            pltpu.VMEM((TBL,), jnp.int32),
            pltpu.VMEM((TBL,), jnp.int32),
            pltpu.VMEM((CHUNK,), jnp.float32),
            pltpu.VMEM((CHUNK,), jnp.float32),
            pltpu.VMEM((CHUNK,), jnp.int32),
            pltpu.VMEM((CHUNK,), jnp.float32),
            pltpu.VMEM((CHUNK * DIM,), jnp.float32),
            pltpu.VMEM((CHUNK * DIM,), jnp.float32),
            pltpu.SemaphoreType.DMA,
            pltpu.SemaphoreType.DMA,
            pltpu.SemaphoreType.DMA,
            pltpu.SemaphoreType.DMA,
            pltpu.SemaphoreType.DMA,
            pltpu.SemaphoreType.DMA,
        ],
    )
    def interp_kernel(t_hbm, tbl_hbm, out_hbm,
                      tbl_a, tbl_b, t_a, t_b, l_v, a_v, o_a, o_b,
                      stbl_a, stbl_b, st_a, st_b, so_a, so_b):
        wid = lax.axis_index("s") * 2 + lax.axis_index("c")
        lane_iota = lax.iota(jnp.int32, LANES)
        hi_iota = lane_iota + LANES
        tbls = (tbl_a, tbl_b)
        ts = (t_a, t_b)
        os_ = (o_a, o_b)
        stbls = (stbl_a, stbl_b)
        sts = (st_a, st_b)
        sos = (so_a, so_b)

        def unit_of(s):
            u = s * NUM_WORKERS + wid
            j = u // N_CHUNKS
            c0 = (u % N_CHUNKS) * CHUNK
            return j, c0

        def prefetch(s, b):
            j, c0 = unit_of(s)
            pltpu.async_copy(tbl_hbm.at[pl.ds(j * TBL, TBL)], tbls[b], stbls[b])
            pltpu.async_copy(t_hbm.at[pl.ds(c0, CHUNK)], ts[b], sts[b])

        def run_unit(s, b, do_drain, do_prefetch):
            j, c0 = unit_of(s)
            tbl_v, t_v, o_v = tbls[b], ts[b], os_[b]

            pltpu.make_async_copy(
                tbl_hbm.at[pl.ds(j * TBL, TBL)], tbl_v, stbls[b]).wait()
            pltpu.make_async_copy(
                t_hbm.at[pl.ds(c0, CHUNK)], t_v, sts[b]).wait()

            if isinstance(do_prefetch, bool):
                if do_prefetch:
                    prefetch(s + 1, 1 - b)
            else:
                @pl.when(do_prefetch)
                def _prefetch_next():
                    prefetch(s + 1, 1 - b)

            @plsc.parallel_loop(0, CHUNK, LANES, unroll=2)
            def idx_body(base):
                tv = t_v[pl.ds(base, LANES)]
                fi = tv * float(N_CP)
                li = fi.astype(jnp.int32)
                li = jnp.minimum(li, N_CP - 1)
                a = fi - li.astype(jnp.float32)
                l_v[pl.ds(base, LANES)] = li * TROW
                a_v[pl.ds(base, LANES)] = a

            @pl.when(do_drain)
            def _drain():
                pltpu.make_async_copy(
                    o_v,
                    out_hbm.at[pl.ds((j * BATCH + c0) * DIM, CHUNK * DIM)],
                    sos[b]).wait()

            @plsc.parallel_loop(0, CHUNK, LANES, unroll=2)
            def grp_body(base):
                lvec = l_v[pl.ds(base, LANES)]
                avec = a_v[pl.ds(base, LANES)]
                for e in range(LANES):
                    lsp = _splat(lvec, e)
                    asp = _splat(avec, e)
                    cw = plsc.load_gather(tbl_v, [lsp + lane_iota])
                    dw = plsc.load_gather(tbl_v, [lsp + hi_iota])
                    cp0, cp1 = plsc.unpack(
                        plsc.bitcast(cw, jnp.bfloat16),
                        format=plsc.PackFormat.INTERLEAVED,
                        preferred_element_type=jnp.float32)
                    d0, d1 = plsc.unpack(
                        plsc.bitcast(dw, jnp.bfloat16),
                        format=plsc.PackFormat.INTERLEAVED,
                        preferred_element_type=jnp.float32)
                    o = base * DIM + e * DIM
                    o_v[pl.ds(o, LANES)] = cp0 + asp * d0
                    o_v[pl.ds(o + LANES, LANES)] = cp1 + asp * d1

            pltpu.async_copy(
                o_v, out_hbm.at[pl.ds((j * BATCH + c0) * DIM, CHUNK * DIM)],
                sos[b])

        prefetch(0, 0)

        def pair_body(s2, carry):
            s0 = s2 * 2
            run_unit(s0, 0, s2 > 0, True)
            run_unit(s0 + 1, 1, s2 > 0, s0 + 2 < UNITS_PER_W)
            return carry

        lax.fori_loop(0, UNITS_PER_W // 2, pair_body, 0)

        for o_v, sem in ((o_a, so_a), (o_b, so_b)):
            pltpu.make_async_copy(
                o_v, out_hbm.at[pl.ds(0, CHUNK * DIM)], sem).wait()

    return interp_kernel


_INTERP = _build_kernel()


def kernel(t, control_points):
    cpt = jnp.swapaxes(control_points, 0, 1)
    delta = jnp.concatenate(
        [cpt[:, 1:, :] - cpt[:, :-1, :],
         jnp.zeros((N_J, 1, DIM), jnp.float32)], axis=1)
    tbl = jnp.concatenate([_pack_pairs(cpt), _pack_pairs(delta)], axis=-1)
    out_flat = _INTERP(t.reshape(BATCH), tbl.reshape(N_J * TBL))
    return out_flat.reshape(N_J, BATCH, DIM)
